# baseline (speedup 1.0000x reference)
import functools
import math

import jax
import jax.numpy as jnp
from jax import lax
from jax.experimental import pallas as pl
from jax.experimental.pallas import tpu as pltpu
from jax.experimental.pallas import tpu_sc as plsc

D_MODEL = 64
MAX_REL = 30
_NUM_CORES = 2
_NUM_SUBCORES = 16
_NUM_WORKERS = _NUM_CORES * _NUM_SUBCORES
_RUNBUF = 512
_SBW = 384


def _tables(q_len: int):
    positions = jnp.arange(0, 2 * MAX_REL + 1, dtype=jnp.float32)[:, None]
    div_term = jnp.exp(
        jnp.arange(0, D_MODEL, 2, dtype=jnp.float32)
        * -(math.log(10000.0) / D_MODEL)
    )
    enc = jnp.zeros((2 * MAX_REL + 1, D_MODEL), dtype=jnp.float32)
    enc = enc.at[:, 0::2].set(jnp.sin(positions * div_term))
    enc = enc.at[:, 1::2].set(jnp.cos(positions * div_term))
    t = jnp.arange(2 * q_len - 1)
    idx = jnp.clip(t - (q_len - 1), -MAX_REL, MAX_REL) + MAX_REL
    ext_t = jnp.take(enc, idx, axis=0).T

    base = q_len - 129
    slib = jnp.stack(
        [lax.slice_in_dim(ext_t, base - p, base - p + _SBW, axis=1)
         for p in range(128)]
    )
    b0row = jnp.broadcast_to(enc[0][:, None], (D_MODEL, _RUNBUF))
    b60row = jnp.broadcast_to(enc[2 * MAX_REL][:, None], (D_MODEL, _RUNBUF))
    return slib, b0row, b60row


def kernel(q):
    q_len = q.shape[2]
    dtype = q.dtype
    slib, b0row, b60row = _tables(q_len)
    slib = slib.astype(dtype)
    b0row = b0row.astype(dtype)
    b60row = b60row.astype(dtype)

    mesh = plsc.VectorSubcoreMesh(core_axis_name="c", subcore_axis_name="s")

    @functools.partial(
        pl.kernel,
        out_type=jax.ShapeDtypeStruct((q_len, D_MODEL, q_len), dtype),
        mesh=mesh,
        scratch_types=[
            pltpu.VMEM((D_MODEL, _RUNBUF), dtype),
            pltpu.VMEM((D_MODEL, _RUNBUF), dtype),
            pltpu.VMEM((2, D_MODEL, _SBW), dtype),
            pltpu.SemaphoreType.DMA,
            pltpu.SemaphoreType.DMA,
        ],
    )
    def sc_fill(slib_hbm, b0_hbm, b60_hbm, out_hbm,
                b0buf, b60buf, mbuf, sem_in, sem_out):
        wid = lax.axis_index("s") * _NUM_CORES + lax.axis_index("c")
        p0 = wid * 4
        pltpu.sync_copy(b0_hbm, b0buf)
        pltpu.sync_copy(b60_hbm, b60buf)

        def emit_runs(i, buf, lo, hi, ds):
            off = lo
            while off < hi:
                w = min(_RUNBUF, hi - off)
                ds.append(pltpu.async_copy(
                    buf.at[:, pl.ds(0, w)],
                    out_hbm.at[i, :, pl.ds(off, w)], sem_out))
                off += w

        stages = {0: pltpu.async_copy(slib_hbm.at[p0], mbuf.at[0], sem_in)}
        bg = []
        mixed = {}
        for pi in range(4):
            p = p0 + pi
            b = pi % 2
            stages[pi].wait()
            mixed[pi] = []
            for k in range(8):
                i = p + 128 * k
                if k == 0:
                    src, off, w = mbuf.at[b, :, pl.ds(128, 256)], 0, 256
                elif k == 7:
                    src, off, w = mbuf.at[b, :, pl.ds(0, 256)], 768, 256
                else:
                    src, off, w = mbuf.at[b], 128 * (k - 1), _SBW
                mixed[pi].append(pltpu.async_copy(
                    src, out_hbm.at[i, :, pl.ds(off, w)], sem_out))
                emit_runs(i, b0buf, 0, off, bg)
                emit_runs(i, b60buf, off + w, q_len, bg)
            if pi < 3:
                if pi >= 1:
                    for d in mixed[pi - 1]:
                        d.wait()
                stages[pi + 1] = pltpu.async_copy(
                    slib_hbm.at[p + 1], mbuf.at[(pi + 1) % 2], sem_in)
        for pi in (2, 3):
            for d in mixed[pi]:
                d.wait()
        for d in bg:
            d.wait()

    out = sc_fill(slib, b0row, b60row)
    return jnp.transpose(out, (0, 2, 1))[None]


# speedup vs baseline: 1.0180x; 1.0180x over previous
import functools
import math

import jax
import jax.numpy as jnp
from jax import lax
from jax.experimental import pallas as pl
from jax.experimental.pallas import tpu as pltpu
from jax.experimental.pallas import tpu_sc as plsc

D_MODEL = 64
MAX_REL = 30
_NUM_CORES = 2
_NUM_SUBCORES = 16
_NUM_WORKERS = _NUM_CORES * _NUM_SUBCORES
_RUNBUF = 384
_SBW = 384


def _tables(q_len: int):
    positions = jnp.arange(0, 2 * MAX_REL + 1, dtype=jnp.float32)[:, None]
    div_term = jnp.exp(
        jnp.arange(0, D_MODEL, 2, dtype=jnp.float32)
        * -(math.log(10000.0) / D_MODEL)
    )
    enc = jnp.zeros((2 * MAX_REL + 1, D_MODEL), dtype=jnp.float32)
    enc = enc.at[:, 0::2].set(jnp.sin(positions * div_term))
    enc = enc.at[:, 1::2].set(jnp.cos(positions * div_term))
    t = jnp.arange(2 * q_len - 1)
    idx = jnp.clip(t - (q_len - 1), -MAX_REL, MAX_REL) + MAX_REL
    ext_t = jnp.take(enc, idx, axis=0).T

    base = q_len - 129
    slib = jnp.stack(
        [lax.slice_in_dim(ext_t, base - p, base - p + _SBW, axis=1)
         for p in range(128)]
    )
    b0row = jnp.broadcast_to(enc[0][:, None], (D_MODEL, _RUNBUF))
    b60row = jnp.broadcast_to(enc[2 * MAX_REL][:, None], (D_MODEL, _RUNBUF))
    return slib, b0row, b60row


def kernel(q):
    q_len = q.shape[2]
    dtype = q.dtype
    slib, b0row, b60row = _tables(q_len)
    slib = slib.astype(dtype)
    b0row = b0row.astype(dtype)
    b60row = b60row.astype(dtype)

    mesh = plsc.VectorSubcoreMesh(core_axis_name="c", subcore_axis_name="s")

    @functools.partial(
        pl.kernel,
        out_type=jax.ShapeDtypeStruct((q_len, D_MODEL, q_len), dtype),
        mesh=mesh,
        scratch_types=[
            pltpu.VMEM((D_MODEL, _RUNBUF), dtype),
            pltpu.VMEM((D_MODEL, _RUNBUF), dtype),
            pltpu.VMEM((3, D_MODEL, _SBW), dtype),
            pltpu.SemaphoreType.DMA,
            pltpu.SemaphoreType.DMA,
        ],
    )
    def sc_fill(slib_hbm, b0_hbm, b60_hbm, out_hbm,
                b0buf, b60buf, mbuf, sem_in, sem_out):
        wid = lax.axis_index("s") * _NUM_CORES + lax.axis_index("c")
        p0 = wid * 4
        pltpu.sync_copy(b0_hbm, b0buf)
        pltpu.sync_copy(b60_hbm, b60buf)

        def emit_runs(i, buf, lo, hi, ds):
            off = lo
            while off < hi:
                w = min(_RUNBUF, hi - off)
                ds.append(pltpu.async_copy(
                    buf.at[:, pl.ds(0, w)],
                    out_hbm.at[i, :, pl.ds(off, w)], sem_out))
                off += w

        stages = {0: pltpu.async_copy(slib_hbm.at[p0], mbuf.at[0], sem_in)}
        bg = []
        mixed = {}
        for pi in range(4):
            p = p0 + pi
            b = pi % 3
            stages[pi].wait()
            if pi < 3:
                if pi >= 2:
                    for d in mixed[pi - 2]:
                        d.wait()
                stages[pi + 1] = pltpu.async_copy(
                    slib_hbm.at[p + 1], mbuf.at[(pi + 1) % 3], sem_in)
            mixed[pi] = []
            for k in range(8):
                i = p + 128 * k
                if k == 0:
                    src, off, w = mbuf.at[b, :, pl.ds(128, 256)], 0, 256
                elif k == 7:
                    src, off, w = mbuf.at[b, :, pl.ds(0, 256)], 768, 256
                else:
                    src, off, w = mbuf.at[b], 128 * (k - 1), _SBW
                mixed[pi].append(pltpu.async_copy(
                    src, out_hbm.at[i, :, pl.ds(off, w)], sem_out))
                emit_runs(i, b0buf, 0, off, bg)
                emit_runs(i, b60buf, off + w, q_len, bg)
        for pi in (1, 2, 3):
            for d in mixed[pi]:
                d.wait()
        for d in bg:
            d.wait()

    out = sc_fill(slib, b0row, b60row)
    return jnp.transpose(out, (0, 2, 1))[None]


# speedup vs baseline: 1.0414x; 1.0229x over previous
import functools
import math

import jax
import jax.numpy as jnp
from jax import lax
from jax.experimental import pallas as pl
from jax.experimental.pallas import tpu as pltpu
from jax.experimental.pallas import tpu_sc as plsc

D_MODEL = 64
MAX_REL = 30
_NUM_CORES = 2
_NUM_SUBCORES = 16
_NUM_WORKERS = _NUM_CORES * _NUM_SUBCORES
_RUNBUF = 384
_SBW = 384


def _tables(q_len: int):
    positions = jnp.arange(0, 2 * MAX_REL + 1, dtype=jnp.float32)[:, None]
    div_term = jnp.exp(
        jnp.arange(0, D_MODEL, 2, dtype=jnp.float32)
        * -(math.log(10000.0) / D_MODEL)
    )
    enc = jnp.zeros((2 * MAX_REL + 1, D_MODEL), dtype=jnp.float32)
    enc = enc.at[:, 0::2].set(jnp.sin(positions * div_term))
    enc = enc.at[:, 1::2].set(jnp.cos(positions * div_term))
    t = jnp.arange(2 * q_len - 1)
    idx = jnp.clip(t - (q_len - 1), -MAX_REL, MAX_REL) + MAX_REL
    ext_t = jnp.take(enc, idx, axis=0).T

    base = q_len - 129
    slib = jnp.stack(
        [lax.slice_in_dim(ext_t, base - p, base - p + _SBW, axis=1)
         for p in range(128)]
    )
    b0row = jnp.broadcast_to(enc[0][:, None], (D_MODEL, _RUNBUF))
    b60row = jnp.broadcast_to(enc[2 * MAX_REL][:, None], (D_MODEL, _RUNBUF))
    return slib, b0row, b60row


def kernel(q):
    q_len = q.shape[2]
    dtype = q.dtype
    slib, b0row, b60row = _tables(q_len)
    slib = slib.astype(dtype)
    b0row = b0row.astype(dtype)
    b60row = b60row.astype(dtype)

    mesh = plsc.VectorSubcoreMesh(core_axis_name="c", subcore_axis_name="s")

    @functools.partial(
        pl.kernel,
        out_type=jax.ShapeDtypeStruct((q_len, D_MODEL, q_len), dtype),
        mesh=mesh,
        scratch_types=[
            pltpu.VMEM((D_MODEL, _RUNBUF), dtype),
            pltpu.VMEM((D_MODEL, _RUNBUF), dtype),
            pltpu.VMEM((3, D_MODEL, _SBW), dtype),
            pltpu.SemaphoreType.DMA,
            pltpu.SemaphoreType.DMA,
        ],
    )
    def sc_fill(slib_hbm, b0_hbm, b60_hbm, out_hbm,
                b0buf, b60buf, mbuf, sem_in, sem_out):
        wid = lax.axis_index("s") * _NUM_CORES + lax.axis_index("c")
        p0 = wid * 4
        run_stages = [
            pltpu.async_copy(b0_hbm, b0buf, sem_in),
            pltpu.async_copy(b60_hbm, b60buf, sem_in),
        ]

        def emit_runs(i, buf, lo, hi, ds):
            off = lo
            while off < hi:
                w = min(_RUNBUF, hi - off)
                ds.append(pltpu.async_copy(
                    buf.at[:, pl.ds(0, w)],
                    out_hbm.at[i, :, pl.ds(off, w)], sem_out))
                off += w

        stages = {0: pltpu.async_copy(slib_hbm.at[p0], mbuf.at[0], sem_in)}
        for d in run_stages:
            d.wait()
        bg = []
        mixed = {}
        for pi in range(4):
            p = p0 + pi
            b = pi % 3
            stages[pi].wait()
            if pi < 3:
                if pi >= 2:
                    for d in mixed[pi - 2]:
                        d.wait()
                stages[pi + 1] = pltpu.async_copy(
                    slib_hbm.at[p + 1], mbuf.at[(pi + 1) % 3], sem_in)
            mixed[pi] = []
            for k in range(8):
                i = p + 128 * k
                if k == 0:
                    src, off, w = mbuf.at[b, :, pl.ds(128, 256)], 0, 256
                elif k == 7:
                    src, off, w = mbuf.at[b, :, pl.ds(0, 256)], 768, 256
                else:
                    src, off, w = mbuf.at[b], 128 * (k - 1), _SBW
                mixed[pi].append(pltpu.async_copy(
                    src, out_hbm.at[i, :, pl.ds(off, w)], sem_out))
                emit_runs(i, b0buf, 0, off, bg)
                emit_runs(i, b60buf, off + w, q_len, bg)
        for pi in (1, 2, 3):
            for d in mixed[pi]:
                d.wait()
        for d in bg:
            d.wait()

    out = sc_fill(slib, b0row, b60row)
    return jnp.transpose(out, (0, 2, 1))[None]
